# baseline (speedup 1.0000x reference)
import jax
import jax.numpy as jnp
from jax import lax
from jax.experimental import pallas as pl
from jax.experimental.pallas import tpu as pltpu
from jax.experimental.pallas import tpu_sc as plsc

B, T, D = 4096, 200, 128
NC, NS = 2, 16
NW = NC * NS
BPW = B // NW
BB = 256


def _tc_idx_body(mask_ref, idx_ref):
    pid = pl.program_id(0)
    m = mask_ref[:, 0, :]
    s = jnp.sum(m, axis=1, keepdims=True)
    b = pid * BB + lax.broadcasted_iota(jnp.int32, (BB, 1), 0)
    row = jnp.where(s == 0, T - 1, s - 1)
    idx_ref[...] = b * T + row


def _sc_gather_body(x_hbm, idx_hbm, out_hbm, idx_v, rows_v, sem):
    wid = lax.axis_index("s") * NC + lax.axis_index("c")
    base = wid * BPW
    pltpu.sync_copy(idx_hbm.at[pl.ds(base, BPW)], idx_v)
    pltpu.async_copy(x_hbm.at[idx_v], rows_v, sem).wait()
    pltpu.sync_copy(rows_v, out_hbm.at[pl.ds(base, BPW)])


def kernel(x, dim, mask):
    del dim
    idx2d = pl.pallas_call(
        _tc_idx_body,
        grid=(B // BB,),
        in_specs=[pl.BlockSpec((BB, 1, T), lambda i: (i, 0, 0))],
        out_specs=pl.BlockSpec((BB, 1), lambda i: (i, 0)),
        out_shape=jax.ShapeDtypeStruct((B, 1), jnp.int32),
    )(mask)
    idx = idx2d.reshape(B)

    mesh = plsc.VectorSubcoreMesh(core_axis_name="c", subcore_axis_name="s")
    run = pl.kernel(
        _sc_gather_body,
        out_type=jax.ShapeDtypeStruct((B, D), jnp.float32),
        mesh=mesh,
        scratch_types=[
            pltpu.VMEM((BPW,), jnp.int32),
            pltpu.VMEM((BPW, D), jnp.float32),
            pltpu.SemaphoreType.DMA,
        ],
    )
    out = run(x.reshape(B * T, D), idx)
    return out.reshape(B, 1, D)


# speedup vs baseline: 1.8120x; 1.8120x over previous
import jax
import jax.numpy as jnp
from jax import lax
from jax.experimental import pallas as pl
from jax.experimental.pallas import tpu as pltpu
from jax.experimental.pallas import tpu_sc as plsc

B, T, D = 4096, 200, 128
NC, NS = 2, 16
NW = NC * NS
BPW = B // NW
LANES = 16
NG = BPW // LANES


def _sc_body(x_hbm, maskT_hbm, out_hbm, maskT_v, idx_v, rows_v, sem):
    wid = lax.axis_index("s") * NC + lax.axis_index("c")
    base = wid * BPW
    pltpu.sync_copy(maskT_hbm.at[0, :, pl.ds(base, BPW)], maskT_v)

    lane = lax.iota(jnp.int32, LANES)

    def t_body(t, accs):
        return tuple(
            acc + maskT_v[t, pl.ds(g * LANES, LANES)]
            for g, acc in enumerate(accs)
        )

    zeros = jnp.zeros((LANES,), jnp.int32)
    sums = lax.fori_loop(0, T, t_body, (zeros,) * NG)

    for g in range(NG):
        s = sums[g]
        row = jnp.where(s == 0, T - 1, s - 1)
        idx_v[pl.ds(g * LANES, LANES)] = (base + g * LANES + lane) * T + row

    pltpu.async_copy(x_hbm.at[idx_v], rows_v, sem).wait()
    pltpu.sync_copy(rows_v, out_hbm.at[pl.ds(base, BPW)])


def kernel(x, dim, mask):
    del dim
    maskT = jnp.transpose(mask, (1, 2, 0))
    mesh = plsc.VectorSubcoreMesh(core_axis_name="c", subcore_axis_name="s")
    run = pl.kernel(
        _sc_body,
        out_type=jax.ShapeDtypeStruct((B, D), jnp.float32),
        mesh=mesh,
        scratch_types=[
            pltpu.VMEM((T, BPW), jnp.int32),
            pltpu.VMEM((BPW,), jnp.int32),
            pltpu.VMEM((BPW, D), jnp.float32),
            pltpu.SemaphoreType.DMA,
        ],
    )
    out = run(x.reshape(B * T, D), maskT)
    return out.reshape(B, 1, D)
